# baseline (speedup 1.0000x reference)
import functools

import jax
import jax.numpy as jnp
from jax import lax
from jax.experimental import pallas as pl
from jax.experimental.pallas import tpu as pltpu
from jax.experimental.pallas import tpu_sc as plsc

VOCAB = 1000000
DIM = 64
BATCH = 1024
GROUP = 10
SEQ = 20
N_DOC = 2
N_USER = 2

DOC_PAIRS = BATCH * GROUP
SEGS = N_DOC * DOC_PAIRS + N_USER * BATCH
NW = 32
PAIRS_W = DOC_PAIRS // NW
UB_W = BATCH // NW
PC = 16
N_DOC_CHUNKS = PAIRS_W // PC
N_USER_CHUNKS = UB_W // PC
IDS_PER_FIELD = PC * SEQ
IDS_PER_CHUNK = 2 * IDS_PER_FIELD
GATHER_SLICE = 128
N_GATHERS = IDS_PER_CHUNK // GATHER_SLICE
LANES = 16
NVREG = DIM // LANES
LENS_PAD = 16

L_DOC0, L_DOC1, L_USR0, L_USR1 = 0, PAIRS_W, 2 * PAIRS_W, 2 * PAIRS_W + UB_W


N_CHUNKS = N_DOC_CHUNKS + N_USER_CHUNKS


def _sc_body(dids_hbm, dlens_hbm, uids_hbm, ulens_hbm, table_hbm, out_hbm,
             idx0, idx1, rows0, rows1, lens_v, out_v, sem0, sem1):
    c = lax.axis_index("c")
    s = lax.axis_index("s")
    wid = s * 2 + c

    pltpu.sync_copy(dlens_hbm.at[pl.ds(wid * PAIRS_W, PAIRS_W)],
                    lens_v.at[pl.ds(L_DOC0, PAIRS_W)])
    pltpu.sync_copy(dlens_hbm.at[pl.ds(DOC_PAIRS + wid * PAIRS_W, PAIRS_W)],
                    lens_v.at[pl.ds(L_DOC1, PAIRS_W)])
    pltpu.sync_copy(ulens_hbm.at[pl.ds(wid * UB_W, UB_W)],
                    lens_v.at[pl.ds(L_USR0, UB_W)])
    pltpu.sync_copy(ulens_hbm.at[pl.ds(BATCH + wid * UB_W, UB_W)],
                    lens_v.at[pl.ds(L_USR1, UB_W)])

    idxs, rowss, sems = (idx0, idx1), (rows0, rows1), (sem0, sem1)

    def chunk_params(ci):
        if ci < N_DOC_CHUNKS:
            pair_base = wid * PAIRS_W + ci * PC
            return (dids_hbm, pair_base * SEQ,
                    (DOC_PAIRS + pair_base) * SEQ,
                    L_DOC0 + ci * PC, L_DOC1 + ci * PC,
                    pair_base * 2 * DIM)
        uci = ci - N_DOC_CHUNKS
        b_base = wid * UB_W + uci * PC
        return (uids_hbm, b_base * SEQ,
                (BATCH + b_base) * SEQ,
                L_USR0 + uci * PC, L_USR1 + uci * PC,
                (N_DOC * DOC_PAIRS + b_base * 2) * DIM)

    def issue(ci):
        b = ci % 2
        idx_v, rows_v, sem = idxs[b], rowss[b], sems[b]
        ids_hbm, id0_off, id1_off, _, _, _ = chunk_params(ci)
        pltpu.sync_copy(ids_hbm.at[pl.ds(id0_off, IDS_PER_FIELD)],
                        idx_v.at[pl.ds(0, IDS_PER_FIELD)])
        pltpu.sync_copy(ids_hbm.at[pl.ds(id1_off, IDS_PER_FIELD)],
                        idx_v.at[pl.ds(IDS_PER_FIELD, IDS_PER_FIELD)])

        def remap(g, _):
            v = idx_v[pl.ds(g * LANES, LANES)]
            a_o = ((v >> 14) << 13) | (v & (TBLK - 1))
            idx_v[pl.ds(g * LANES, LANES)] = (a_o << 1) | ((v >> 13) & 1)
            return ()

        lax.fori_loop(0, IDS_PER_CHUNK // LANES, remap, ())

        return [
            pltpu.async_copy(
                table_hbm.at[idx_v.at[pl.ds(k * GATHER_SLICE, GATHER_SLICE)]],
                rows_v.at[pl.ds(k * GATHER_SLICE, GATHER_SLICE), :],
                sem,
            )
            for k in range(N_GATHERS)
        ]

    def accumulate(ci):
        rows_v = rowss[ci % 2]
        _, _, _, len0_off, len1_off, out_off = chunk_params(ci)

        def seg_body(k16, _):
            for f, len_off in ((0, len0_off), (1, len1_off)):
                lv = lens_v[pl.ds(len_off + k16, LANES)]
                ln = lv[0]
                invv = 1.0 / jnp.maximum(lv.astype(jnp.float32), 1.0)
                inv = jnp.where(ln > 0, invv[0], 0.0)
                row0 = f * PC * SEQ + k16 * SEQ

                def row_body(l, accs):
                    return tuple(
                        accs[d] + rows_v[row0 + l, pl.ds(d * LANES, LANES)]
                        for d in range(NVREG))

                accs = lax.fori_loop(
                    0, ln,
                    row_body,
                    tuple(jnp.zeros((LANES,), jnp.float32)
                          for _ in range(NVREG)))
                obase = (2 * k16 + f) * DIM
                for d in range(NVREG):
                    out_v[pl.ds(obase + d * LANES, LANES)] = accs[d] * inv
            return ()

        lax.fori_loop(0, PC, seg_body, ())
        pltpu.sync_copy(out_v, out_hbm.at[pl.ds(out_off, 2 * PC * DIM)])

    handles = issue(0)
    for ci in range(N_CHUNKS):
        nxt = issue(ci + 1) if ci + 1 < N_CHUNKS else []
        for h in handles:
            h.wait()
        accumulate(ci)
        handles = nxt


TBLK = 8192
N_TBLK = -(-VOCAB // (2 * TBLK))
TROWS = N_TBLK * TBLK
TVIEW_ROWS = 2 * TROWS


def _tc_transpose_body(src_ref, dst_ref):
    x = src_ref[...]
    xe = jnp.transpose(x[:, :TBLK])
    xo = jnp.transpose(x[:, TBLK:])
    dst_ref[...] = jnp.concatenate([xe, xo], axis=1)


def _linearize_table(table):
    table_t = jnp.transpose(table)
    packed = pl.pallas_call(
        _tc_transpose_body,
        grid=(N_TBLK,),
        in_specs=[pl.BlockSpec((DIM, 2 * TBLK), lambda i: (0, i))],
        out_specs=pl.BlockSpec((TBLK, 2 * DIM), lambda i: (i, 0)),
        out_shape=jax.ShapeDtypeStruct((TROWS, 2 * DIM), jnp.float32),
    )(table_t)
    return packed.reshape(TVIEW_ROWS, DIM)


@jax.jit
def _run(dids, dlens, uids, ulens, table):
    table = _linearize_table(table)
    mesh = plsc.VectorSubcoreMesh(core_axis_name="c", subcore_axis_name="s")
    kern = functools.partial(
        pl.kernel,
        mesh=mesh,
        out_type=jax.ShapeDtypeStruct((SEGS * DIM,), jnp.float32),
        scratch_types=[
            pltpu.VMEM((IDS_PER_CHUNK,), jnp.int32),
            pltpu.VMEM((IDS_PER_CHUNK,), jnp.int32),
            pltpu.VMEM((IDS_PER_CHUNK, DIM), jnp.float32),
            pltpu.VMEM((IDS_PER_CHUNK, DIM), jnp.float32),
            pltpu.VMEM((2 * (PAIRS_W + UB_W) + LENS_PAD,), jnp.int32),
            pltpu.VMEM((2 * PC * DIM,), jnp.float32),
            pltpu.SemaphoreType.DMA,
            pltpu.SemaphoreType.DMA,
        ],
        compiler_params=pltpu.CompilerParams(use_tc_tiling_on_sc=False),
    )(_sc_body)
    return kern(dids, dlens, uids, ulens, table)


def kernel(doc_ids, doc_len, user_ids, user_len, table):
    out = _run(
        doc_ids.reshape(-1),
        doc_len.reshape(-1),
        user_ids.reshape(-1),
        user_len.reshape(-1),
        table,
    )
    n_doc_rows = N_DOC * DOC_PAIRS
    doc_ftrs = out[: n_doc_rows * DIM].reshape(BATCH, GROUP, N_DOC, DIM)
    user_ftrs = out[n_doc_rows * DIM:].reshape(BATCH, N_USER, DIM)
    return (doc_ftrs, user_ftrs)


# speedup vs baseline: 1.0110x; 1.0110x over previous
import functools

import jax
import jax.numpy as jnp
from jax import lax
from jax.experimental import pallas as pl
from jax.experimental.pallas import tpu as pltpu
from jax.experimental.pallas import tpu_sc as plsc

VOCAB = 1000000
DIM = 64
BATCH = 1024
GROUP = 10
SEQ = 20
N_DOC = 2
N_USER = 2

DOC_PAIRS = BATCH * GROUP
SEGS = N_DOC * DOC_PAIRS + N_USER * BATCH
NW = 32
PAIRS_W = DOC_PAIRS // NW
UB_W = BATCH // NW
PC = 16
N_DOC_CHUNKS = PAIRS_W // PC
N_USER_CHUNKS = UB_W // PC
IDS_PER_FIELD = PC * SEQ
IDS_PER_CHUNK = 2 * IDS_PER_FIELD
GATHER_SLICE = 128
N_GATHERS = IDS_PER_CHUNK // GATHER_SLICE
LANES = 16
NVREG = DIM // LANES
LENS_PAD = 16

L_DOC0, L_DOC1, L_USR0, L_USR1 = 0, PAIRS_W, 2 * PAIRS_W, 2 * PAIRS_W + UB_W


N_CHUNKS = N_DOC_CHUNKS + N_USER_CHUNKS


def _sc_body(dids_hbm, dlens_hbm, uids_hbm, ulens_hbm, table_hbm, out_hbm,
             idx0, idx1, rows0, rows1, lens_v, out_v, sem0, sem1):
    c = lax.axis_index("c")
    s = lax.axis_index("s")
    wid = s * 2 + c

    pltpu.sync_copy(dlens_hbm.at[pl.ds(wid * PAIRS_W, PAIRS_W)],
                    lens_v.at[pl.ds(L_DOC0, PAIRS_W)])
    pltpu.sync_copy(dlens_hbm.at[pl.ds(DOC_PAIRS + wid * PAIRS_W, PAIRS_W)],
                    lens_v.at[pl.ds(L_DOC1, PAIRS_W)])
    pltpu.sync_copy(ulens_hbm.at[pl.ds(wid * UB_W, UB_W)],
                    lens_v.at[pl.ds(L_USR0, UB_W)])
    pltpu.sync_copy(ulens_hbm.at[pl.ds(BATCH + wid * UB_W, UB_W)],
                    lens_v.at[pl.ds(L_USR1, UB_W)])

    idxs, rowss, sems = (idx0, idx1), (rows0, rows1), (sem0, sem1)

    def chunk_params(ci):
        if ci < N_DOC_CHUNKS:
            pair_base = wid * PAIRS_W + ci * PC
            return (dids_hbm, pair_base * SEQ,
                    (DOC_PAIRS + pair_base) * SEQ,
                    L_DOC0 + ci * PC, L_DOC1 + ci * PC,
                    pair_base * 2 * DIM)
        uci = ci - N_DOC_CHUNKS
        b_base = wid * UB_W + uci * PC
        return (uids_hbm, b_base * SEQ,
                (BATCH + b_base) * SEQ,
                L_USR0 + uci * PC, L_USR1 + uci * PC,
                (N_DOC * DOC_PAIRS + b_base * 2) * DIM)

    def issue(ci):
        b = ci % 2
        idx_v, rows_v, sem = idxs[b], rowss[b], sems[b]
        ids_hbm, id0_off, id1_off, _, _, _ = chunk_params(ci)
        pltpu.sync_copy(ids_hbm.at[pl.ds(id0_off, IDS_PER_FIELD)],
                        idx_v.at[pl.ds(0, IDS_PER_FIELD)])
        pltpu.sync_copy(ids_hbm.at[pl.ds(id1_off, IDS_PER_FIELD)],
                        idx_v.at[pl.ds(IDS_PER_FIELD, IDS_PER_FIELD)])

        def remap(g, _):
            v = idx_v[pl.ds(g * LANES, LANES)]
            a_o = ((v >> 14) << 13) | (v & (TBLK - 1))
            idx_v[pl.ds(g * LANES, LANES)] = (a_o << 1) | ((v >> 13) & 1)
            return ()

        lax.fori_loop(0, IDS_PER_CHUNK // LANES, remap, ())

        return [
            pltpu.async_copy(
                table_hbm.at[idx_v.at[pl.ds(k * GATHER_SLICE, GATHER_SLICE)]],
                rows_v.at[pl.ds(k * GATHER_SLICE, GATHER_SLICE), :],
                sem,
            )
            for k in range(N_GATHERS)
        ]

    def accumulate(ci):
        rows_v = rowss[ci % 2]
        _, _, _, len0_off, len1_off, out_off = chunk_params(ci)

        def seg_body(k16, _):
            for f, len_off in ((0, len0_off), (1, len1_off)):
                lv = lens_v[pl.ds(len_off + k16, LANES)]
                ln = lv[0]
                invv = 1.0 / jnp.maximum(lv.astype(jnp.float32), 1.0)
                inv = jnp.where(ln > 0, invv[0], 0.0)
                accs = [jnp.zeros((LANES,), jnp.float32)
                        for _ in range(NVREG)]
                row0 = f * PC * SEQ + k16 * SEQ
                for l in range(SEQ):
                    w = jnp.where(l < ln, inv, 0.0)
                    for d in range(NVREG):
                        r = rows_v[row0 + l, pl.ds(d * LANES, LANES)]
                        accs[d] = accs[d] + r * w
                obase = (2 * k16 + f) * DIM
                for d in range(NVREG):
                    out_v[pl.ds(obase + d * LANES, LANES)] = accs[d]
            return ()

        lax.fori_loop(0, PC, seg_body, ())
        pltpu.sync_copy(out_v, out_hbm.at[pl.ds(out_off, 2 * PC * DIM)])

    handles = issue(0)
    for ci in range(N_CHUNKS):
        nxt = issue(ci + 1) if ci + 1 < N_CHUNKS else []
        for h in handles:
            h.wait()
        accumulate(ci)
        handles = nxt


TBLK = 8192
N_TBLK = -(-VOCAB // (2 * TBLK))
TROWS = N_TBLK * TBLK
TVIEW_ROWS = 2 * TROWS


def _tc_transpose_body(src_ref, dst_ref):
    x = src_ref[...]
    eye = jnp.eye(DIM, dtype=jnp.float32)
    dn = (((0,), (0,)), ((), ()))
    dst_ref[:, :DIM] = lax.dot_general(
        x[:, :TBLK], eye, dn, preferred_element_type=jnp.float32)
    dst_ref[:, DIM:] = lax.dot_general(
        x[:, TBLK:], eye, dn, preferred_element_type=jnp.float32)


def _linearize_table(table):
    table_t = jnp.transpose(table)
    packed = pl.pallas_call(
        _tc_transpose_body,
        grid=(N_TBLK,),
        in_specs=[pl.BlockSpec((DIM, 2 * TBLK), lambda i: (0, i))],
        out_specs=pl.BlockSpec((TBLK, 2 * DIM), lambda i: (i, 0)),
        out_shape=jax.ShapeDtypeStruct((TROWS, 2 * DIM), jnp.float32),
    )(table_t)
    return packed.reshape(TVIEW_ROWS, DIM)


@jax.jit
def _run(dids, dlens, uids, ulens, table):
    table = _linearize_table(table)
    mesh = plsc.VectorSubcoreMesh(core_axis_name="c", subcore_axis_name="s")
    kern = functools.partial(
        pl.kernel,
        mesh=mesh,
        out_type=jax.ShapeDtypeStruct((SEGS * DIM,), jnp.float32),
        scratch_types=[
            pltpu.VMEM((IDS_PER_CHUNK,), jnp.int32),
            pltpu.VMEM((IDS_PER_CHUNK,), jnp.int32),
            pltpu.VMEM((IDS_PER_CHUNK, DIM), jnp.float32),
            pltpu.VMEM((IDS_PER_CHUNK, DIM), jnp.float32),
            pltpu.VMEM((2 * (PAIRS_W + UB_W) + LENS_PAD,), jnp.int32),
            pltpu.VMEM((2 * PC * DIM,), jnp.float32),
            pltpu.SemaphoreType.DMA,
            pltpu.SemaphoreType.DMA,
        ],
        compiler_params=pltpu.CompilerParams(use_tc_tiling_on_sc=False),
    )(_sc_body)
    return kern(dids, dlens, uids, ulens, table)


def kernel(doc_ids, doc_len, user_ids, user_len, table):
    out = _run(
        doc_ids.reshape(-1),
        doc_len.reshape(-1),
        user_ids.reshape(-1),
        user_len.reshape(-1),
        table,
    )
    n_doc_rows = N_DOC * DOC_PAIRS
    doc_ftrs = out[: n_doc_rows * DIM].reshape(BATCH, GROUP, N_DOC, DIM)
    user_ftrs = out[n_doc_rows * DIM:].reshape(BATCH, N_USER, DIM)
    return (doc_ftrs, user_ftrs)


# speedup vs baseline: 1.0123x; 1.0013x over previous
import functools

import jax
import jax.numpy as jnp
from jax import lax
from jax.experimental import pallas as pl
from jax.experimental.pallas import tpu as pltpu
from jax.experimental.pallas import tpu_sc as plsc

VOCAB = 1000000
DIM = 64
BATCH = 1024
GROUP = 10
SEQ = 20
N_DOC = 2
N_USER = 2

DOC_PAIRS = BATCH * GROUP
SEGS = N_DOC * DOC_PAIRS + N_USER * BATCH
NW = 32
PAIRS_W = DOC_PAIRS // NW
UB_W = BATCH // NW
PC = 16
N_DOC_CHUNKS = PAIRS_W // PC
N_USER_CHUNKS = UB_W // PC
IDS_PER_FIELD = PC * SEQ
IDS_PER_CHUNK = 2 * IDS_PER_FIELD
GATHER_SLICE = 128
N_GATHERS = IDS_PER_CHUNK // GATHER_SLICE
LANES = 16
NVREG = DIM // LANES
LENS_PAD = 16

L_DOC0, L_DOC1, L_USR0, L_USR1 = 0, PAIRS_W, 2 * PAIRS_W, 2 * PAIRS_W + UB_W


N_CHUNKS = N_DOC_CHUNKS + N_USER_CHUNKS


def _sc_body(dids_hbm, dlens_hbm, uids_hbm, ulens_hbm, table_hbm, out_hbm,
             idx0, idx1, rows0, rows1, lens_v, out_v, sem0, sem1):
    c = lax.axis_index("c")
    s = lax.axis_index("s")
    wid = s * 2 + c

    pltpu.sync_copy(dlens_hbm.at[pl.ds(wid * PAIRS_W, PAIRS_W)],
                    lens_v.at[pl.ds(L_DOC0, PAIRS_W)])
    pltpu.sync_copy(dlens_hbm.at[pl.ds(DOC_PAIRS + wid * PAIRS_W, PAIRS_W)],
                    lens_v.at[pl.ds(L_DOC1, PAIRS_W)])
    pltpu.sync_copy(ulens_hbm.at[pl.ds(wid * UB_W, UB_W)],
                    lens_v.at[pl.ds(L_USR0, UB_W)])
    pltpu.sync_copy(ulens_hbm.at[pl.ds(BATCH + wid * UB_W, UB_W)],
                    lens_v.at[pl.ds(L_USR1, UB_W)])

    idxs, rowss, sems = (idx0, idx1), (rows0, rows1), (sem0, sem1)

    def zero_idx(g, _):
        z = jnp.zeros((LANES,), jnp.int32)
        idx0[pl.ds(g * LANES, LANES)] = z
        idx1[pl.ds(g * LANES, LANES)] = z
        return ()

    lax.fori_loop(0, IDS_PER_CHUNK // LANES, zero_idx, ())

    def chunk_params(ci):
        if ci < N_DOC_CHUNKS:
            pair_base = wid * PAIRS_W + ci * PC
            return (dids_hbm, pair_base * SEQ,
                    (DOC_PAIRS + pair_base) * SEQ,
                    L_DOC0 + ci * PC, L_DOC1 + ci * PC,
                    pair_base * 2 * DIM)
        uci = ci - N_DOC_CHUNKS
        b_base = wid * UB_W + uci * PC
        return (uids_hbm, b_base * SEQ,
                (BATCH + b_base) * SEQ,
                L_USR0 + uci * PC, L_USR1 + uci * PC,
                (N_DOC * DOC_PAIRS + b_base * 2) * DIM)

    def issue(ci):
        b = ci % 2
        idx_v, rows_v, sem = idxs[b], rowss[b], sems[b]
        ids_hbm, id0_off, id1_off, _, _, _ = chunk_params(ci)
        pltpu.sync_copy(ids_hbm.at[pl.ds(id0_off, IDS_PER_FIELD)],
                        idx_v.at[pl.ds(0, IDS_PER_FIELD)])
        pltpu.sync_copy(ids_hbm.at[pl.ds(id1_off, IDS_PER_FIELD)],
                        idx_v.at[pl.ds(IDS_PER_FIELD, IDS_PER_FIELD)])

        def remap(g, _):
            v = idx_v[pl.ds(g * LANES, LANES)]
            a_o = ((v >> 14) << 13) | (v & (TBLK - 1))
            idx_v[pl.ds(g * LANES, LANES)] = (a_o << 1) | ((v >> 13) & 1)
            return ()

        lax.fori_loop(0, IDS_PER_CHUNK // LANES, remap, ())

        return [
            pltpu.async_copy(
                table_hbm.at[idx_v.at[pl.ds(k * GATHER_SLICE, GATHER_SLICE)]],
                rows_v.at[pl.ds(k * GATHER_SLICE, GATHER_SLICE), :],
                sem,
            )
            for k in range(N_GATHERS)
        ]

    def accumulate(ci):
        rows_v = rowss[ci % 2]
        _, _, _, len0_off, len1_off, out_off = chunk_params(ci)

        def seg_body(k16, _):
            for f, len_off in ((0, len0_off), (1, len1_off)):
                lv = lens_v[pl.ds(len_off + k16, LANES)]
                ln = lv[0]
                invv = 1.0 / jnp.maximum(lv.astype(jnp.float32), 1.0)
                inv = jnp.where(ln > 0, invv[0], 0.0)
                accs = [jnp.zeros((LANES,), jnp.float32)
                        for _ in range(NVREG)]
                row0 = f * PC * SEQ + k16 * SEQ
                for l in range(SEQ):
                    w = jnp.where(l < ln, inv, 0.0)
                    for d in range(NVREG):
                        r = rows_v[row0 + l, pl.ds(d * LANES, LANES)]
                        accs[d] = accs[d] + r * w
                obase = (2 * k16 + f) * DIM
                for d in range(NVREG):
                    out_v[pl.ds(obase + d * LANES, LANES)] = accs[d]
            return ()

        lax.fori_loop(0, PC, seg_body, ())
        pltpu.sync_copy(out_v, out_hbm.at[pl.ds(out_off, 2 * PC * DIM)])

    handles = issue(0)
    for ci in range(N_CHUNKS):
        nxt = issue(ci + 1) if ci + 1 < N_CHUNKS else []
        for h in handles:
            h.wait()
        accumulate(ci)
        handles = nxt


TBLK = 8192
N_TBLK = -(-VOCAB // (2 * TBLK))
TROWS = N_TBLK * TBLK
TVIEW_ROWS = 2 * TROWS


def _tc_transpose_body(src_ref, dst_ref):
    x = src_ref[...]
    dst_ref[:, :DIM] = jnp.transpose(x[:, :TBLK])
    dst_ref[:, DIM:] = jnp.transpose(x[:, TBLK:])


def _linearize_table(table):
    table_t = jnp.transpose(table)
    packed = pl.pallas_call(
        _tc_transpose_body,
        grid=(N_TBLK,),
        in_specs=[pl.BlockSpec((DIM, 2 * TBLK), lambda i: (0, i))],
        out_specs=pl.BlockSpec((TBLK, 2 * DIM), lambda i: (i, 0)),
        out_shape=jax.ShapeDtypeStruct((TROWS, 2 * DIM), jnp.float32),
    )(table_t)
    return packed.reshape(TVIEW_ROWS, DIM)


@jax.jit
def _run(dids, dlens, uids, ulens, table):
    table = _linearize_table(table)
    mesh = plsc.VectorSubcoreMesh(core_axis_name="c", subcore_axis_name="s")
    kern = functools.partial(
        pl.kernel,
        mesh=mesh,
        out_type=jax.ShapeDtypeStruct((SEGS * DIM,), jnp.float32),
        scratch_types=[
            pltpu.VMEM((IDS_PER_CHUNK,), jnp.int32),
            pltpu.VMEM((IDS_PER_CHUNK,), jnp.int32),
            pltpu.VMEM((IDS_PER_CHUNK, DIM), jnp.float32),
            pltpu.VMEM((IDS_PER_CHUNK, DIM), jnp.float32),
            pltpu.VMEM((2 * (PAIRS_W + UB_W) + LENS_PAD,), jnp.int32),
            pltpu.VMEM((2 * PC * DIM,), jnp.float32),
            pltpu.SemaphoreType.DMA,
            pltpu.SemaphoreType.DMA,
        ],
        compiler_params=pltpu.CompilerParams(use_tc_tiling_on_sc=False),
    )(_sc_body)
    return kern(dids, dlens, uids, ulens, table)


def kernel(doc_ids, doc_len, user_ids, user_len, table):
    out = _run(
        doc_ids.reshape(-1),
        doc_len.reshape(-1),
        user_ids.reshape(-1),
        user_len.reshape(-1),
        table,
    )
    n_doc_rows = N_DOC * DOC_PAIRS
    doc_ftrs = out[: n_doc_rows * DIM].reshape(BATCH, GROUP, N_DOC, DIM)
    user_ftrs = out[n_doc_rows * DIM:].reshape(BATCH, N_USER, DIM)
    return (doc_ftrs, user_ftrs)


# speedup vs baseline: 1.0137x; 1.0014x over previous
import functools

import jax
import jax.numpy as jnp
from jax import lax
from jax.experimental import pallas as pl
from jax.experimental.pallas import tpu as pltpu
from jax.experimental.pallas import tpu_sc as plsc

VOCAB = 1000000
DIM = 64
BATCH = 1024
GROUP = 10
SEQ = 20
N_DOC = 2
N_USER = 2

DOC_PAIRS = BATCH * GROUP
SEGS = N_DOC * DOC_PAIRS + N_USER * BATCH
NW = 32
PAIRS_W = DOC_PAIRS // NW
UB_W = BATCH // NW
PC = 16
N_DOC_CHUNKS = PAIRS_W // PC
N_USER_CHUNKS = UB_W // PC
IDS_PER_FIELD = PC * SEQ
IDS_PER_CHUNK = 2 * IDS_PER_FIELD
GATHER_SLICE = 128
N_GATHERS = IDS_PER_CHUNK // GATHER_SLICE
LANES = 16
NVREG = DIM // LANES
LENS_PAD = 16

L_DOC0, L_DOC1, L_USR0, L_USR1 = 0, PAIRS_W, 2 * PAIRS_W, 2 * PAIRS_W + UB_W


N_CHUNKS = N_DOC_CHUNKS + N_USER_CHUNKS


def _sc_body(dids_hbm, dlens_hbm, uids_hbm, ulens_hbm, table_hbm, out_hbm,
             idx0, idx1, idx2, rows0, rows1, rows2, lens_v, out_v,
             sem0, sem1, sem2):
    c = lax.axis_index("c")
    s = lax.axis_index("s")
    wid = s * 2 + c

    pltpu.sync_copy(dlens_hbm.at[pl.ds(wid * PAIRS_W, PAIRS_W)],
                    lens_v.at[pl.ds(L_DOC0, PAIRS_W)])
    pltpu.sync_copy(dlens_hbm.at[pl.ds(DOC_PAIRS + wid * PAIRS_W, PAIRS_W)],
                    lens_v.at[pl.ds(L_DOC1, PAIRS_W)])
    pltpu.sync_copy(ulens_hbm.at[pl.ds(wid * UB_W, UB_W)],
                    lens_v.at[pl.ds(L_USR0, UB_W)])
    pltpu.sync_copy(ulens_hbm.at[pl.ds(BATCH + wid * UB_W, UB_W)],
                    lens_v.at[pl.ds(L_USR1, UB_W)])

    idxs = (idx0, idx1, idx2)
    rowss = (rows0, rows1, rows2)
    sems = (sem0, sem1, sem2)
    DEPTH = 3

    def zero_idx(g, _):
        z = jnp.zeros((LANES,), jnp.int32)
        idx0[pl.ds(g * LANES, LANES)] = z
        idx1[pl.ds(g * LANES, LANES)] = z
        idx2[pl.ds(g * LANES, LANES)] = z
        return ()

    lax.fori_loop(0, IDS_PER_CHUNK // LANES, zero_idx, ())

    def chunk_params(ci):
        if ci < N_DOC_CHUNKS:
            pair_base = wid * PAIRS_W + ci * PC
            return (dids_hbm, pair_base * SEQ,
                    (DOC_PAIRS + pair_base) * SEQ,
                    L_DOC0 + ci * PC, L_DOC1 + ci * PC,
                    pair_base * 2 * DIM)
        uci = ci - N_DOC_CHUNKS
        b_base = wid * UB_W + uci * PC
        return (uids_hbm, b_base * SEQ,
                (BATCH + b_base) * SEQ,
                L_USR0 + uci * PC, L_USR1 + uci * PC,
                (N_DOC * DOC_PAIRS + b_base * 2) * DIM)

    def issue(ci):
        b = ci % DEPTH
        idx_v, rows_v, sem = idxs[b], rowss[b], sems[b]
        ids_hbm, id0_off, id1_off, _, _, _ = chunk_params(ci)
        pltpu.sync_copy(ids_hbm.at[pl.ds(id0_off, IDS_PER_FIELD)],
                        idx_v.at[pl.ds(0, IDS_PER_FIELD)])
        pltpu.sync_copy(ids_hbm.at[pl.ds(id1_off, IDS_PER_FIELD)],
                        idx_v.at[pl.ds(IDS_PER_FIELD, IDS_PER_FIELD)])

        def remap(g, _):
            v = idx_v[pl.ds(g * LANES, LANES)]
            a_o = ((v >> 14) << 13) | (v & (TBLK - 1))
            idx_v[pl.ds(g * LANES, LANES)] = (a_o << 1) | ((v >> 13) & 1)
            return ()

        lax.fori_loop(0, IDS_PER_CHUNK // LANES, remap, ())

        return [
            pltpu.async_copy(
                table_hbm.at[idx_v.at[pl.ds(k * GATHER_SLICE, GATHER_SLICE)]],
                rows_v.at[pl.ds(k * GATHER_SLICE, GATHER_SLICE), :],
                sem,
            )
            for k in range(N_GATHERS)
        ]

    def accumulate(ci):
        rows_v = rowss[ci % DEPTH]
        _, _, _, len0_off, len1_off, out_off = chunk_params(ci)

        def seg_body(k16, _):
            for f, len_off in ((0, len0_off), (1, len1_off)):
                lv = lens_v[pl.ds(len_off + k16, LANES)]
                ln = lv[0]
                invv = 1.0 / jnp.maximum(lv.astype(jnp.float32), 1.0)
                inv = jnp.where(ln > 0, invv[0], 0.0)
                accs = [jnp.zeros((LANES,), jnp.float32)
                        for _ in range(NVREG)]
                row0 = f * PC * SEQ + k16 * SEQ
                for l in range(SEQ):
                    w = jnp.where(l < ln, inv, 0.0)
                    for d in range(NVREG):
                        r = rows_v[row0 + l, pl.ds(d * LANES, LANES)]
                        accs[d] = accs[d] + r * w
                obase = (2 * k16 + f) * DIM
                for d in range(NVREG):
                    out_v[pl.ds(obase + d * LANES, LANES)] = accs[d]
            return ()

        lax.fori_loop(0, PC, seg_body, ())
        pltpu.sync_copy(out_v, out_hbm.at[pl.ds(out_off, 2 * PC * DIM)])

    pending = [issue(0), issue(1)]
    for ci in range(N_CHUNKS):
        nxt = issue(ci + 2) if ci + 2 < N_CHUNKS else []
        for h in pending.pop(0):
            h.wait()
        accumulate(ci)
        pending.append(nxt)


TBLK = 8192
N_TBLK = -(-VOCAB // (2 * TBLK))
TROWS = N_TBLK * TBLK
TVIEW_ROWS = 2 * TROWS


def _tc_transpose_body(src_ref, dst_ref):
    x = src_ref[...]
    dst_ref[:, :DIM] = jnp.transpose(x[:, :TBLK])
    dst_ref[:, DIM:] = jnp.transpose(x[:, TBLK:])


def _linearize_table(table):
    table_t = jnp.transpose(table)
    packed = pl.pallas_call(
        _tc_transpose_body,
        grid=(N_TBLK,),
        in_specs=[pl.BlockSpec((DIM, 2 * TBLK), lambda i: (0, i))],
        out_specs=pl.BlockSpec((TBLK, 2 * DIM), lambda i: (i, 0)),
        out_shape=jax.ShapeDtypeStruct((TROWS, 2 * DIM), jnp.float32),
    )(table_t)
    return packed.reshape(TVIEW_ROWS, DIM)


@jax.jit
def _run(dids, dlens, uids, ulens, table):
    table = _linearize_table(table)
    mesh = plsc.VectorSubcoreMesh(core_axis_name="c", subcore_axis_name="s")
    kern = functools.partial(
        pl.kernel,
        mesh=mesh,
        out_type=jax.ShapeDtypeStruct((SEGS * DIM,), jnp.float32),
        scratch_types=[
            pltpu.VMEM((IDS_PER_CHUNK,), jnp.int32),
            pltpu.VMEM((IDS_PER_CHUNK,), jnp.int32),
            pltpu.VMEM((IDS_PER_CHUNK,), jnp.int32),
            pltpu.VMEM((IDS_PER_CHUNK, DIM), jnp.float32),
            pltpu.VMEM((IDS_PER_CHUNK, DIM), jnp.float32),
            pltpu.VMEM((IDS_PER_CHUNK, DIM), jnp.float32),
            pltpu.VMEM((2 * (PAIRS_W + UB_W) + LENS_PAD,), jnp.int32),
            pltpu.VMEM((2 * PC * DIM,), jnp.float32),
            pltpu.SemaphoreType.DMA,
            pltpu.SemaphoreType.DMA,
            pltpu.SemaphoreType.DMA,
        ],
        compiler_params=pltpu.CompilerParams(use_tc_tiling_on_sc=False),
    )(_sc_body)
    return kern(dids, dlens, uids, ulens, table)


def kernel(doc_ids, doc_len, user_ids, user_len, table):
    out = _run(
        doc_ids.reshape(-1),
        doc_len.reshape(-1),
        user_ids.reshape(-1),
        user_len.reshape(-1),
        table,
    )
    n_doc_rows = N_DOC * DOC_PAIRS
    doc_ftrs = out[: n_doc_rows * DIM].reshape(BATCH, GROUP, N_DOC, DIM)
    user_ftrs = out[n_doc_rows * DIM:].reshape(BATCH, N_USER, DIM)
    return (doc_ftrs, user_ftrs)
